# baseline (speedup 1.0000x reference)
import jax
import jax.numpy as jnp
from jax import lax
from jax.experimental import pallas as pl
from jax.experimental.pallas import tpu as pltpu
from jax.experimental.pallas import tpu_sc as plsc

D = 64
NC = 2
NS = 16
NW = NC * NS
CHUNK = 128


def _emb_body(idx_hbm, table_hbm, out_hbm, idx_v, rows_v, gsem):
    wid = lax.axis_index("s") * NC + lax.axis_index("c")
    nch = idx_hbm.shape[1]
    pltpu.sync_copy(idx_hbm.at[wid], idx_v)

    @pl.loop(0, nch)
    def _(g):
        pltpu.async_copy(table_hbm.at[idx_v.at[g]], rows_v, gsem).wait()
        pltpu.sync_copy(rows_v, out_hbm.at[wid, g])


def kernel(idx, weight):
    B, F = idx.shape
    total = B * F
    b_per_w = total // NW
    nch = b_per_w // CHUNK
    idx3 = idx.reshape(NW, nch, CHUNK).astype(jnp.int32)
    k = pl.kernel(
        _emb_body,
        out_type=jax.ShapeDtypeStruct((NW, nch, CHUNK, D), jnp.float32),
        mesh=plsc.VectorSubcoreMesh(core_axis_name="c", subcore_axis_name="s"),
        scratch_types=[
            pltpu.VMEM((nch, CHUNK), jnp.int32),
            pltpu.VMEM((CHUNK, D), jnp.float32),
            pltpu.SemaphoreType.DMA,
        ],
        compiler_params=pltpu.CompilerParams(use_tc_tiling_on_sc=False),
    )
    out = k(idx3, weight)
    return out.reshape(B, F, D)


# speedup vs baseline: 1.0758x; 1.0758x over previous
import jax
import jax.numpy as jnp
from jax import lax
from jax.experimental import pallas as pl
from jax.experimental.pallas import tpu as pltpu
from jax.experimental.pallas import tpu_sc as plsc

D = 64
NC = 2
NS = 16
NW = NC * NS
CHUNK = 128


NBUF = 8


def _emb_body(idx_hbm, table_hbm, out_hbm, idx_v, rows_v, gsem, osem):
    wid = lax.axis_index("s") * NC + lax.axis_index("c")
    nch = idx_hbm.shape[1]
    pltpu.sync_copy(idx_hbm.at[wid], idx_v)

    for b in range(NBUF):
        pltpu.async_copy(table_hbm.at[idx_v.at[b]], rows_v.at[b], gsem.at[b])

    @pl.loop(0, nch - NBUF, step=NBUF)
    def _(go):
        for b in range(NBUF):
            g = go + b
            pltpu.make_async_copy(
                table_hbm.at[idx_v.at[g]], rows_v.at[b], gsem.at[b]).wait()
            pltpu.async_copy(rows_v.at[b], out_hbm.at[wid, g], osem.at[b])
            pltpu.make_async_copy(
                rows_v.at[b], out_hbm.at[wid, g], osem.at[b]).wait()
            pltpu.async_copy(
                table_hbm.at[idx_v.at[g + NBUF]], rows_v.at[b], gsem.at[b])

    for b in range(NBUF):
        g = nch - NBUF + b
        pltpu.make_async_copy(
            table_hbm.at[idx_v.at[g]], rows_v.at[b], gsem.at[b]).wait()
        pltpu.sync_copy(rows_v.at[b], out_hbm.at[wid, g])


def kernel(idx, weight):
    B, F = idx.shape
    total = B * F
    b_per_w = total // NW
    nch = b_per_w // CHUNK
    idx3 = idx.reshape(NW, nch, CHUNK).astype(jnp.int32)
    k = pl.kernel(
        _emb_body,
        out_type=jax.ShapeDtypeStruct((NW, nch, CHUNK, D), jnp.float32),
        mesh=plsc.VectorSubcoreMesh(core_axis_name="c", subcore_axis_name="s"),
        scratch_types=[
            pltpu.VMEM((nch, CHUNK), jnp.int32),
            pltpu.VMEM((NBUF, CHUNK, D), jnp.float32),
            pltpu.SemaphoreType.DMA((NBUF,)),
            pltpu.SemaphoreType.DMA((NBUF,)),
        ],
        compiler_params=pltpu.CompilerParams(use_tc_tiling_on_sc=False),
    )
    out = k(idx3, weight)
    return out.reshape(B, F, D)
